# baseline (device time: 89937 ns/iter reference)
import jax
import jax.numpy as jnp
from jax import lax
from jax.experimental import pallas as pl
from jax.experimental.pallas import tpu as pltpu


def kernel(ids, E):
    T = ids.shape[0]
    V_shard, D = E.shape
    Th = T // 2

    my_x = lax.axis_index("x")
    my_y = lax.axis_index("y")

    ids_half = lax.dynamic_slice(ids, (my_y * Th,), (Th,))
    loc = ids_half - my_x * V_shard
    in_range = (loc >= 0) & (loc < V_shard)
    rows = jnp.take(E, jnp.where(in_range, loc, 0), axis=0)
    part = (rows * in_range[:, None].astype(E.dtype)).astype(jnp.bfloat16)

    def body(part_ref, out_ref, xbuf, red, ybuf, send_sems, recv_sems):
        x = lax.axis_index("x")
        y = lax.axis_index("y")
        ox = 1 - x
        oy = 1 - y

        barrier = pltpu.get_barrier_semaphore()
        pl.semaphore_signal(barrier, inc=1, device_id=(ox, y),
                            device_id_type=pl.DeviceIdType.MESH)
        pl.semaphore_signal(barrier, inc=1, device_id=(x, oy),
                            device_id_type=pl.DeviceIdType.MESH)
        pl.semaphore_wait(barrier, 2)

        rdma1 = pltpu.make_async_remote_copy(
            src_ref=part_ref,
            dst_ref=xbuf,
            send_sem=send_sems.at[0],
            recv_sem=recv_sems.at[0],
            device_id=(ox, y),
            device_id_type=pl.DeviceIdType.MESH,
        )
        rdma1.start()
        rdma1.wait()

        red[...] = part_ref[...] + xbuf[...]
        out_ref[pl.ds(y * Th, Th), :] = red[...].astype(jnp.float32)

        rdma2 = pltpu.make_async_remote_copy(
            src_ref=red,
            dst_ref=ybuf,
            send_sem=send_sems.at[1],
            recv_sem=recv_sems.at[1],
            device_id=(x, oy),
            device_id_type=pl.DeviceIdType.MESH,
        )
        rdma2.start()
        rdma2.wait()
        out_ref[pl.ds(oy * Th, Th), :] = ybuf[...].astype(jnp.float32)

    return pl.pallas_call(
        body,
        out_shape=jax.ShapeDtypeStruct((T, D), jnp.float32),
        in_specs=[pl.BlockSpec(memory_space=pltpu.VMEM)],
        out_specs=pl.BlockSpec(memory_space=pltpu.VMEM),
        scratch_shapes=[
            pltpu.VMEM((Th, D), jnp.bfloat16),
            pltpu.VMEM((Th, D), jnp.bfloat16),
            pltpu.VMEM((Th, D), jnp.bfloat16),
            pltpu.SemaphoreType.DMA((2,)),
            pltpu.SemaphoreType.DMA((2,)),
        ],
        compiler_params=pltpu.CompilerParams(collective_id=0),
    )(part)


# device time: 69831 ns/iter; 1.2879x vs baseline; 1.2879x over previous
import jax
import jax.numpy as jnp
from jax import lax
from jax.experimental import pallas as pl
from jax.experimental.pallas import tpu as pltpu

C = 8


def kernel(ids, E):
    T = ids.shape[0]
    V_shard, D = E.shape
    Th = T // 2
    R = Th // C

    my_x = lax.axis_index("x")
    my_y = lax.axis_index("y")

    ids_half = lax.dynamic_slice(ids, (my_y * Th,), (Th,))
    loc = ids_half - my_x * V_shard
    in_range = (loc >= 0) & (loc < V_shard)
    rows = jnp.take(E, jnp.where(in_range, loc, 0), axis=0)
    part = (rows * in_range[:, None].astype(E.dtype)).astype(jnp.bfloat16)

    def body(part_ref, out_ref, xbuf, red, ybuf, s1, r1, s2, r2):
        x = lax.axis_index("x")
        y = lax.axis_index("y")
        ox = 1 - x
        oy = 1 - y

        barrier = pltpu.get_barrier_semaphore()
        pl.semaphore_signal(barrier, inc=1, device_id=(ox, y),
                            device_id_type=pl.DeviceIdType.MESH)
        pl.semaphore_signal(barrier, inc=1, device_id=(x, oy),
                            device_id_type=pl.DeviceIdType.MESH)
        pl.semaphore_wait(barrier, 2)

        def chunk(ref, c):
            return ref.at[pl.ds(c * R, R), :]

        rdma1 = [
            pltpu.make_async_remote_copy(
                src_ref=chunk(part_ref, c),
                dst_ref=chunk(xbuf, c),
                send_sem=s1.at[c],
                recv_sem=r1.at[c],
                device_id=(ox, y),
                device_id_type=pl.DeviceIdType.MESH,
            )
            for c in range(C)
        ]
        rdma2 = [
            pltpu.make_async_remote_copy(
                src_ref=chunk(red, c),
                dst_ref=chunk(ybuf, c),
                send_sem=s2.at[c],
                recv_sem=r2.at[c],
                device_id=(x, oy),
                device_id_type=pl.DeviceIdType.MESH,
            )
            for c in range(C)
        ]

        for c in range(C):
            rdma1[c].start()

        for c in range(C):
            rdma1[c].wait_recv()
            red[pl.ds(c * R, R), :] = (
                part_ref[pl.ds(c * R, R), :] + xbuf[pl.ds(c * R, R), :]
            )
            rdma2[c].start()
            out_ref[pl.ds(y * Th + c * R, R), :] = (
                red[pl.ds(c * R, R), :].astype(jnp.float32)
            )
            rdma1[c].wait_send()

        for c in range(C):
            rdma2[c].wait_recv()
            out_ref[pl.ds(oy * Th + c * R, R), :] = (
                ybuf[pl.ds(c * R, R), :].astype(jnp.float32)
            )
            rdma2[c].wait_send()

    return pl.pallas_call(
        body,
        out_shape=jax.ShapeDtypeStruct((T, D), jnp.float32),
        in_specs=[pl.BlockSpec(memory_space=pltpu.VMEM)],
        out_specs=pl.BlockSpec(memory_space=pltpu.VMEM),
        scratch_shapes=[
            pltpu.VMEM((Th, D), jnp.bfloat16),
            pltpu.VMEM((Th, D), jnp.bfloat16),
            pltpu.VMEM((Th, D), jnp.bfloat16),
            pltpu.SemaphoreType.DMA((C,)),
            pltpu.SemaphoreType.DMA((C,)),
            pltpu.SemaphoreType.DMA((C,)),
            pltpu.SemaphoreType.DMA((C,)),
        ],
        compiler_params=pltpu.CompilerParams(collective_id=0),
    )(part)


# device time: 61600 ns/iter; 1.4600x vs baseline; 1.1336x over previous
import jax
import jax.numpy as jnp
from jax import lax
from jax.experimental import pallas as pl
from jax.experimental.pallas import tpu as pltpu

C = 8


def kernel(ids, E):
    T = ids.shape[0]
    V_shard, D = E.shape
    Th = T // 2
    R = Th // C

    my_x = lax.axis_index("x")
    my_y = lax.axis_index("y")

    ids_half = lax.dynamic_slice(ids, (my_y * Th,), (Th,))
    loc = ids_half - my_x * V_shard
    in_range = (loc >= 0) & (loc < V_shard)
    sloc = jnp.clip(loc, 0, V_shard - 1)
    mask = in_range[:, None].astype(E.dtype)

    def body(sloc_ref, mask_ref, E_ref, out_ref,
             gbuf, part, xbuf, red, ybuf, gsem, s1, r1, s2, r2):
        x = lax.axis_index("x")
        y = lax.axis_index("y")
        ox = 1 - x
        oy = 1 - y

        def row_copy(row, t, c):
            return pltpu.make_async_copy(
                E_ref.at[pl.ds(row, 1), :],
                gbuf.at[pl.ds(t, 1), :],
                gsem.at[c],
            )

        def gissue(c):
            def issue_row(t, _):
                row = sloc_ref[c * R + t]
                row_copy(row, c * R + t, c).start()
                return 0
            lax.fori_loop(0, R, issue_row, 0)

        def gwait(c):
            def wait_row(t, _):
                row_copy(0, 0, c).wait()
                return 0
            lax.fori_loop(0, R, wait_row, 0)

        def chunk(ref, c):
            return ref.at[pl.ds(c * R, R), :]

        rdma1 = [
            pltpu.make_async_remote_copy(
                src_ref=chunk(part, c),
                dst_ref=chunk(xbuf, c),
                send_sem=s1.at[c],
                recv_sem=r1.at[c],
                device_id=(ox, y),
                device_id_type=pl.DeviceIdType.MESH,
            )
            for c in range(C)
        ]
        rdma2 = [
            pltpu.make_async_remote_copy(
                src_ref=chunk(red, c),
                dst_ref=chunk(ybuf, c),
                send_sem=s2.at[c],
                recv_sem=r2.at[c],
                device_id=(x, oy),
                device_id_type=pl.DeviceIdType.MESH,
            )
            for c in range(C)
        ]

        gissue(0)
        gissue(1)
        barrier = pltpu.get_barrier_semaphore()
        pl.semaphore_signal(barrier, inc=1, device_id=(ox, y),
                            device_id_type=pl.DeviceIdType.MESH)
        pl.semaphore_signal(barrier, inc=1, device_id=(x, oy),
                            device_id_type=pl.DeviceIdType.MESH)
        pl.semaphore_wait(barrier, 2)

        for c in range(C):
            gwait(c)
            part[pl.ds(c * R, R), :] = (
                gbuf[pl.ds(c * R, R), :] * mask_ref[pl.ds(c * R, R), :]
            ).astype(jnp.bfloat16)
            rdma1[c].start()
            if c + 2 < C:
                gissue(c + 2)

        for c in range(C):
            rdma1[c].wait_recv()
            red[pl.ds(c * R, R), :] = (
                part[pl.ds(c * R, R), :] + xbuf[pl.ds(c * R, R), :]
            )
            rdma2[c].start()
            out_ref[pl.ds(y * Th + c * R, R), :] = (
                red[pl.ds(c * R, R), :].astype(jnp.float32)
            )
            rdma1[c].wait_send()

        for c in range(C):
            rdma2[c].wait_recv()
            out_ref[pl.ds(oy * Th + c * R, R), :] = (
                ybuf[pl.ds(c * R, R), :].astype(jnp.float32)
            )
            rdma2[c].wait_send()

    return pl.pallas_call(
        body,
        out_shape=jax.ShapeDtypeStruct((T, D), jnp.float32),
        in_specs=[
            pl.BlockSpec(memory_space=pltpu.SMEM),
            pl.BlockSpec(memory_space=pltpu.VMEM),
            pl.BlockSpec(memory_space=pl.ANY),
        ],
        out_specs=pl.BlockSpec(memory_space=pltpu.VMEM),
        scratch_shapes=[
            pltpu.VMEM((Th, D), jnp.float32),
            pltpu.VMEM((Th, D), jnp.bfloat16),
            pltpu.VMEM((Th, D), jnp.bfloat16),
            pltpu.VMEM((Th, D), jnp.bfloat16),
            pltpu.VMEM((Th, D), jnp.bfloat16),
            pltpu.SemaphoreType.DMA((C,)),
            pltpu.SemaphoreType.DMA((C,)),
            pltpu.SemaphoreType.DMA((C,)),
            pltpu.SemaphoreType.DMA((C,)),
            pltpu.SemaphoreType.DMA((C,)),
        ],
        compiler_params=pltpu.CompilerParams(collective_id=0),
    )(sloc, mask, E)


# device time: 53698 ns/iter; 1.6749x vs baseline; 1.1472x over previous
import jax
import jax.numpy as jnp
from jax import lax
from jax.experimental import pallas as pl
from jax.experimental.pallas import tpu as pltpu

C = 8


def kernel(ids, E):
    T = ids.shape[0]
    V_shard, D = E.shape
    Th = T // 2
    R = Th // C

    my_x = lax.axis_index("x")
    my_y = lax.axis_index("y")

    ids_half = lax.dynamic_slice(ids, (my_y * Th,), (Th,))
    loc = ids_half - my_x * V_shard
    in_range = (loc >= 0) & (loc < V_shard)
    sloc = jnp.clip(loc, 0, V_shard - 1)
    mask = in_range[:, None].astype(E.dtype)

    def body(sloc_ref, mask_ref, E_ref, out_ref,
             gbuf, part, xbuf, red, ybuf, gsem, s1, r1, s2, r2):
        x = lax.axis_index("x")
        y = lax.axis_index("y")
        ox = 1 - x
        oy = 1 - y

        def row_copy(row, t, c):
            return pltpu.make_async_copy(
                E_ref.at[pl.ds(row, 1), :],
                gbuf.at[pl.ds(t, 1), :],
                gsem.at[c],
            )

        def gissue(c):
            def issue_row(t, _):
                row = sloc_ref[c * R + t]
                row_copy(row, c * R + t, c).start()
                return 0
            lax.fori_loop(0, R, issue_row, 0, unroll=8)

        def gwait(c):
            pltpu.make_async_copy(
                E_ref.at[pl.ds(0, R), :],
                gbuf.at[pl.ds(c * R, R), :],
                gsem.at[c],
            ).wait()

        def chunk(ref, c):
            return ref.at[pl.ds(c * R, R), :]

        rdma1 = [
            pltpu.make_async_remote_copy(
                src_ref=chunk(part, c),
                dst_ref=chunk(xbuf, c),
                send_sem=s1.at[c],
                recv_sem=r1.at[c],
                device_id=(ox, y),
                device_id_type=pl.DeviceIdType.MESH,
            )
            for c in range(C)
        ]
        rdma2 = [
            pltpu.make_async_remote_copy(
                src_ref=chunk(red, c),
                dst_ref=chunk(ybuf, c),
                send_sem=s2.at[c],
                recv_sem=r2.at[c],
                device_id=(x, oy),
                device_id_type=pl.DeviceIdType.MESH,
            )
            for c in range(C)
        ]

        gissue(0)
        gissue(1)
        barrier = pltpu.get_barrier_semaphore()
        pl.semaphore_signal(barrier, inc=1, device_id=(ox, y),
                            device_id_type=pl.DeviceIdType.MESH)
        pl.semaphore_signal(barrier, inc=1, device_id=(x, oy),
                            device_id_type=pl.DeviceIdType.MESH)
        pl.semaphore_wait(barrier, 2)

        for c in range(C):
            gwait(c)
            part[pl.ds(c * R, R), :] = (
                gbuf[pl.ds(c * R, R), :] * mask_ref[pl.ds(c * R, R), :]
            ).astype(jnp.bfloat16)
            rdma1[c].start()
            if c + 2 < C:
                gissue(c + 2)

        for c in range(C):
            rdma1[c].wait_recv()
            red[pl.ds(c * R, R), :] = (
                part[pl.ds(c * R, R), :] + xbuf[pl.ds(c * R, R), :]
            )
            rdma2[c].start()
            out_ref[pl.ds(y * Th + c * R, R), :] = (
                red[pl.ds(c * R, R), :].astype(jnp.float32)
            )
            rdma1[c].wait_send()

        for c in range(C):
            rdma2[c].wait_recv()
            out_ref[pl.ds(oy * Th + c * R, R), :] = (
                ybuf[pl.ds(c * R, R), :].astype(jnp.float32)
            )
            rdma2[c].wait_send()

    return pl.pallas_call(
        body,
        out_shape=jax.ShapeDtypeStruct((T, D), jnp.float32),
        in_specs=[
            pl.BlockSpec(memory_space=pltpu.SMEM),
            pl.BlockSpec(memory_space=pltpu.VMEM),
            pl.BlockSpec(memory_space=pl.ANY),
        ],
        out_specs=pl.BlockSpec(memory_space=pltpu.VMEM),
        scratch_shapes=[
            pltpu.VMEM((Th, D), jnp.float32),
            pltpu.VMEM((Th, D), jnp.bfloat16),
            pltpu.VMEM((Th, D), jnp.bfloat16),
            pltpu.VMEM((Th, D), jnp.bfloat16),
            pltpu.VMEM((Th, D), jnp.bfloat16),
            pltpu.SemaphoreType.DMA((C,)),
            pltpu.SemaphoreType.DMA((C,)),
            pltpu.SemaphoreType.DMA((C,)),
            pltpu.SemaphoreType.DMA((C,)),
            pltpu.SemaphoreType.DMA((C,)),
        ],
        compiler_params=pltpu.CompilerParams(collective_id=0),
    )(sloc, mask, E)


# device time: 42991 ns/iter; 2.0920x vs baseline; 1.2491x over previous
import jax
import jax.numpy as jnp
from jax import lax
from jax.experimental import pallas as pl
from jax.experimental.pallas import tpu as pltpu

C = 8


def kernel(ids, E):
    T = ids.shape[0]
    V_shard, D = E.shape
    Th = T // 2
    R = Th // C

    my_x = lax.axis_index("x")
    my_y = lax.axis_index("y")

    ids_half = lax.dynamic_slice(ids, (my_y * Th,), (Th,))
    loc = ids_half - my_x * V_shard
    in_range = (loc >= 0) & (loc < V_shard)
    sloc = jnp.clip(loc, 0, V_shard - 1)
    mask = in_range[:, None].astype(E.dtype)

    def body(sloc_ref, mask_ref, E_ref, out_ref,
             gbuf, part, xbuf, red, ybuf, gsem, s1, r1, s2, r2):
        x = lax.axis_index("x")
        y = lax.axis_index("y")
        ox = 1 - x
        oy = 1 - y

        def gissue(c):
            def issue_row(t, _):
                row = sloc_ref[c * R + t]
                pltpu.make_async_copy(
                    E_ref.at[pl.ds(row, 1), :],
                    gbuf.at[pl.ds(c * R + t, 1), :],
                    gsem.at[c],
                ).start()
                return 0
            lax.fori_loop(0, R, issue_row, 0, unroll=8)

        def gwait(c):
            pltpu.make_async_copy(
                E_ref.at[pl.ds(0, R), :],
                gbuf.at[pl.ds(c * R, R), :],
                gsem.at[c],
            ).wait()

        def chunk(ref, c):
            return ref.at[pl.ds(c * R, R), :]

        rdma1 = [
            pltpu.make_async_remote_copy(
                src_ref=chunk(part, c),
                dst_ref=chunk(xbuf, c),
                send_sem=s1.at[c],
                recv_sem=r1.at[c],
                device_id=(ox, y),
                device_id_type=pl.DeviceIdType.MESH,
            )
            for c in range(C)
        ]
        rdma2 = [
            pltpu.make_async_remote_copy(
                src_ref=chunk(red, c),
                dst_ref=chunk(ybuf, c),
                send_sem=s2.at[c],
                recv_sem=r2.at[c],
                device_id=(x, oy),
                device_id_type=pl.DeviceIdType.MESH,
            )
            for c in range(C)
        ]

        gissue(0)
        gissue(1)
        barrier = pltpu.get_barrier_semaphore()
        pl.semaphore_signal(barrier, inc=1, device_id=(ox, y),
                            device_id_type=pl.DeviceIdType.MESH)
        pl.semaphore_signal(barrier, inc=1, device_id=(x, oy),
                            device_id_type=pl.DeviceIdType.MESH)
        pl.semaphore_wait(barrier, 2)

        for i in range(C + 2):
            if i < C:
                c = i
                gwait(c)
                part[pl.ds(c * R, R), :] = (
                    gbuf[pl.ds(c * R, R), :] * mask_ref[pl.ds(c * R, R), :]
                ).astype(jnp.bfloat16)
                rdma1[c].start()
                if c + 2 < C:
                    gissue(c + 2)
            if 1 <= i <= C:
                c = i - 1
                rdma1[c].wait_recv()
                red[pl.ds(c * R, R), :] = (
                    part[pl.ds(c * R, R), :] + xbuf[pl.ds(c * R, R), :]
                )
                rdma2[c].start()
                out_ref[pl.ds(y * Th + c * R, R), :] = (
                    red[pl.ds(c * R, R), :].astype(jnp.float32)
                )
                rdma1[c].wait_send()
            if i >= 2:
                c = i - 2
                rdma2[c].wait_recv()
                out_ref[pl.ds(oy * Th + c * R, R), :] = (
                    ybuf[pl.ds(c * R, R), :].astype(jnp.float32)
                )
                rdma2[c].wait_send()

    return pl.pallas_call(
        body,
        out_shape=jax.ShapeDtypeStruct((T, D), jnp.float32),
        in_specs=[
            pl.BlockSpec(memory_space=pltpu.SMEM),
            pl.BlockSpec(memory_space=pltpu.VMEM),
            pl.BlockSpec(memory_space=pl.ANY),
        ],
        out_specs=pl.BlockSpec(memory_space=pltpu.VMEM),
        scratch_shapes=[
            pltpu.VMEM((Th, D), jnp.float32),
            pltpu.VMEM((Th, D), jnp.bfloat16),
            pltpu.VMEM((Th, D), jnp.bfloat16),
            pltpu.VMEM((Th, D), jnp.bfloat16),
            pltpu.VMEM((Th, D), jnp.bfloat16),
            pltpu.SemaphoreType.DMA((C,)),
            pltpu.SemaphoreType.DMA((C,)),
            pltpu.SemaphoreType.DMA((C,)),
            pltpu.SemaphoreType.DMA((C,)),
            pltpu.SemaphoreType.DMA((C,)),
        ],
        compiler_params=pltpu.CompilerParams(collective_id=0),
    )(sloc, mask, E)
